# baseline (device time: 101752 ns/iter reference)
import jax
import jax.numpy as jnp
from jax import lax
from jax.experimental import pallas as pl
from jax.experimental.pallas import tpu as pltpu

N_DEV = 4
B_LOC = 2
SQ = 512
SKV = 512
H_LOC = 8
DH = 64
D_MODEL = 768
D_BLOCK = H_LOC * DH

_CompilerParams = getattr(pltpu, "CompilerParams", None) or getattr(
    pltpu, "TPUCompilerParams"
)


def _body(x_ref, wq_ref, k_ref, v_ref, wo_ref, out_ref,
          wq_buf, wo_buf, ctx_scr,
          wq_send, wq_recv, wo_send, wo_recv):
    my = lax.axis_index("i")
    left = (my - 1) % N_DEV
    right = (my + 1) % N_DEV

    bar = pltpu.get_barrier_semaphore()
    pl.semaphore_signal(bar, inc=1, device_id=(left,),
                        device_id_type=pl.DeviceIdType.MESH)
    pl.semaphore_signal(bar, inc=1, device_id=(right,),
                        device_id_type=pl.DeviceIdType.MESH)
    pl.semaphore_wait(bar, 2)

    qi = lax.broadcasted_iota(jnp.int32, (SQ, SKV), 0)
    ki = lax.broadcasted_iota(jnp.int32, (SQ, SKV), 1)
    mask = (jnp.abs(qi - ki) <= 128) | (ki < 32) | (qi < 32)
    bias = jnp.where(mask, 0.0, -1e9).astype(jnp.float32)

    for s in range(N_DEV):
        wq_src = wq_ref if s == 0 else wq_buf.at[s - 1]
        wo_src = wo_ref if s == 0 else wo_buf.at[s - 1]
        if s < N_DEV - 1:
            rd_wq = pltpu.make_async_remote_copy(
                src_ref=wq_src, dst_ref=wq_buf.at[s],
                send_sem=wq_send.at[s], recv_sem=wq_recv.at[s],
                device_id=(right,), device_id_type=pl.DeviceIdType.MESH,
            )
            rd_wq.start()
            rd_wo = pltpu.make_async_remote_copy(
                src_ref=wo_src, dst_ref=wo_buf.at[s],
                send_sem=wo_send.at[s], recv_sem=wo_recv.at[s],
                device_id=(right,), device_id_type=pl.DeviceIdType.MESH,
            )
            rd_wo.start()

        hb = (my - s) % N_DEV
        wq_s = wq_src[...]
        wo_s = wo_src[...]

        for b in range(B_LOC):
            qb = jnp.dot(x_ref[b], wq_s,
                         preferred_element_type=jnp.float32,
                         ).astype(jnp.bfloat16)
            for h in range(H_LOC):
                head = hb * H_LOC + h
                qh = qb[:, h * DH:(h + 1) * DH]
                kh = k_ref[head, b]
                sc = lax.dot_general(
                    qh, kh, (((1,), (1,)), ((), ())),
                    preferred_element_type=jnp.float32)
                e = jnp.exp(sc * 0.125 + bias).astype(jnp.bfloat16)
                r = 1.0 / jnp.sum(e, axis=1, keepdims=True,
                                  dtype=jnp.float32)
                vh = v_ref[head, b]
                ctxh = jnp.dot(e, vh,
                               preferred_element_type=jnp.float32)
                ctx_scr[:, h * DH:(h + 1) * DH] = (
                    ctxh * r).astype(jnp.bfloat16)
            contrib = jnp.dot(ctx_scr[...], wo_s,
                              preferred_element_type=jnp.float32)
            if s == 0:
                out_ref[b] = contrib
            else:
                out_ref[b] = out_ref[b] + contrib

        if s < N_DEV - 1:
            rd_wq.wait()
            rd_wo.wait()


def kernel(x, Wq, K_ext, V_ext, Wo):
    my = lax.axis_index("i")
    Kb = lax.dynamic_slice_in_dim(K_ext, my * B_LOC, B_LOC, axis=0)
    Vb = lax.dynamic_slice_in_dim(V_ext, my * B_LOC, B_LOC, axis=0)
    Kt = jnp.transpose(Kb, (2, 0, 1, 3)).astype(jnp.bfloat16)
    Vt = jnp.transpose(Vb, (2, 0, 1, 3)).astype(jnp.bfloat16)
    x16 = x.astype(jnp.bfloat16)
    Wq16 = Wq.astype(jnp.bfloat16)
    Wo16 = Wo.astype(jnp.bfloat16)

    return pl.pallas_call(
        _body,
        out_shape=jax.ShapeDtypeStruct((B_LOC, SQ, D_MODEL), jnp.float32),
        in_specs=[pl.BlockSpec(memory_space=pltpu.VMEM)] * 5,
        out_specs=pl.BlockSpec(memory_space=pltpu.VMEM),
        scratch_shapes=[
            pltpu.VMEM((N_DEV - 1, D_MODEL, D_BLOCK), jnp.bfloat16),
            pltpu.VMEM((N_DEV - 1, D_BLOCK, D_MODEL), jnp.bfloat16),
            pltpu.VMEM((SQ, D_BLOCK), jnp.bfloat16),
            pltpu.SemaphoreType.DMA((N_DEV - 1,)),
            pltpu.SemaphoreType.DMA((N_DEV - 1,)),
            pltpu.SemaphoreType.DMA((N_DEV - 1,)),
            pltpu.SemaphoreType.DMA((N_DEV - 1,)),
        ],
        compiler_params=_CompilerParams(
            collective_id=0, vmem_limit_bytes=100 * 1024 * 1024),
    )(x16, Wq16, Kt, Vt, Wo16)


# device time: 80293 ns/iter; 1.2673x vs baseline; 1.2673x over previous
import jax
import jax.numpy as jnp
from jax import lax
from jax.experimental import pallas as pl
from jax.experimental.pallas import tpu as pltpu

N_DEV = 4
B_LOC = 2
SQ = 512
SKV = 512
H_LOC = 8
DH = 64
D_MODEL = 768
D_BLOCK = H_LOC * DH
HWQ = D_MODEL // 2
HWO = D_BLOCK // 2

_CompilerParams = getattr(pltpu, "CompilerParams", None) or getattr(
    pltpu, "TPUCompilerParams"
)


def _body(x_ref, wq_ref, k_ref, v_ref, wo_ref, out_ref,
          wq_buf, wo_buf, ctx_scr, ss, rs):
    my = lax.axis_index("i")
    left = (my - 1) % N_DEV
    right = (my + 1) % N_DEV

    def rcopy(src, dst, s_idx, r_idx, dev):
        return pltpu.make_async_remote_copy(
            src_ref=src, dst_ref=dst,
            send_sem=ss.at[s_idx], recv_sem=rs.at[r_idx],
            device_id=(dev,), device_id_type=pl.DeviceIdType.MESH,
        )

    bar = pltpu.get_barrier_semaphore()
    pl.semaphore_signal(bar, inc=1, device_id=(left,),
                        device_id_type=pl.DeviceIdType.MESH)
    pl.semaphore_signal(bar, inc=1, device_id=(right,),
                        device_id_type=pl.DeviceIdType.MESH)
    pl.semaphore_wait(bar, 2)

    sends = [
        rcopy(wq_ref, wq_buf.at[0], 0, 0, right),
        rcopy(wo_ref, wo_buf.at[0], 1, 1, right),
        rcopy(wq_ref, wq_buf.at[1], 2, 2, left),
        rcopy(wo_ref, wo_buf.at[1], 3, 3, left),
    ]
    for r in sends:
        r.start()

    qi = lax.broadcasted_iota(jnp.int32, (SQ, SKV), 0)
    ki = lax.broadcasted_iota(jnp.int32, (SQ, SKV), 1)
    mask = (jnp.abs(qi - ki) <= 128) | (ki < 32) | (qi < 32)
    bias = jnp.where(mask, 0.0, -1e9).astype(jnp.float32)

    def compute(wq_src, wo_src, hb, first):
        wq_s = wq_src[...]
        wo_s = wo_src[...]
        for b in range(B_LOC):
            qb = jnp.dot(x_ref[b], wq_s,
                         preferred_element_type=jnp.float32,
                         ).astype(jnp.bfloat16)
            for h in range(H_LOC):
                head = hb * H_LOC + h
                qh = qb[:, h * DH:(h + 1) * DH]
                kh = k_ref[head, b]
                sc = lax.dot_general(
                    qh, kh, (((1,), (1,)), ((), ())),
                    preferred_element_type=jnp.float32)
                e = jnp.exp(sc * 0.125 + bias).astype(jnp.bfloat16)
                r = 1.0 / jnp.sum(e, axis=1, keepdims=True,
                                  dtype=jnp.float32)
                vh = v_ref[head, b]
                ctxh = jnp.dot(e, vh,
                               preferred_element_type=jnp.float32)
                ctx_scr[:, h * DH:(h + 1) * DH] = (
                    ctxh * r).astype(jnp.bfloat16)
            contrib = jnp.dot(ctx_scr[...], wo_s,
                              preferred_element_type=jnp.float32)
            if first:
                out_ref[b] = contrib
            else:
                out_ref[b] = out_ref[b] + contrib

    compute(wq_ref, wo_ref, my, first=True)

    for r in sends:
        r.wait_recv()
    fwds = [
        rcopy(wq_buf.at[0, pl.ds(0, HWQ), :], wq_buf.at[2, pl.ds(0, HWQ), :],
              4, 4, right),
        rcopy(wo_buf.at[0, pl.ds(0, HWO), :], wo_buf.at[2, pl.ds(0, HWO), :],
              5, 5, right),
        rcopy(wq_buf.at[1, pl.ds(HWQ, HWQ), :], wq_buf.at[2, pl.ds(HWQ, HWQ), :],
              6, 6, left),
        rcopy(wo_buf.at[1, pl.ds(HWO, HWO), :], wo_buf.at[2, pl.ds(HWO, HWO), :],
              7, 7, left),
    ]
    for r in fwds:
        r.start()

    compute(wq_buf.at[0], wo_buf.at[0], (my - 1) % N_DEV, first=False)
    compute(wq_buf.at[1], wo_buf.at[1], (my + 1) % N_DEV, first=False)

    for r in fwds:
        r.wait_recv()
    compute(wq_buf.at[2], wo_buf.at[2], (my + 2) % N_DEV, first=False)

    for r in sends + fwds:
        r.wait_send()


def kernel(x, Wq, K_ext, V_ext, Wo):
    my = lax.axis_index("i")
    Kb = lax.dynamic_slice_in_dim(K_ext, my * B_LOC, B_LOC, axis=0)
    Vb = lax.dynamic_slice_in_dim(V_ext, my * B_LOC, B_LOC, axis=0)
    Kt = jnp.transpose(Kb, (2, 0, 1, 3)).astype(jnp.bfloat16)
    Vt = jnp.transpose(Vb, (2, 0, 1, 3)).astype(jnp.bfloat16)
    x16 = x.astype(jnp.bfloat16)
    Wq16 = Wq.astype(jnp.bfloat16)
    Wo16 = Wo.astype(jnp.bfloat16)

    return pl.pallas_call(
        _body,
        out_shape=jax.ShapeDtypeStruct((B_LOC, SQ, D_MODEL), jnp.float32),
        in_specs=[pl.BlockSpec(memory_space=pltpu.VMEM)] * 5,
        out_specs=pl.BlockSpec(memory_space=pltpu.VMEM),
        scratch_shapes=[
            pltpu.VMEM((N_DEV - 1, D_MODEL, D_BLOCK), jnp.bfloat16),
            pltpu.VMEM((N_DEV - 1, D_BLOCK, D_MODEL), jnp.bfloat16),
            pltpu.VMEM((SQ, D_BLOCK), jnp.bfloat16),
            pltpu.SemaphoreType.DMA((8,)),
            pltpu.SemaphoreType.DMA((8,)),
        ],
        compiler_params=_CompilerParams(
            collective_id=0, vmem_limit_bytes=100 * 1024 * 1024),
    )(x16, Wq16, Kt, Vt, Wo16)
